# baseline (speedup 1.0000x reference)
import functools

import jax
import jax.numpy as jnp
from jax import lax
from jax.experimental import pallas as pl
from jax.experimental.pallas import tpu as pltpu
from jax.experimental.pallas import tpu_sc as plsc

BATCH = 16384
FEAT = 64
LANES = 16
NUM_CORES = 2
NUM_SUBCORES = 16
NUM_WORKERS = NUM_CORES * NUM_SUBCORES
BPW = BATCH // NUM_WORKERS
CHUNK = 128
NCHUNKS = BPW // CHUNK
VECS_PER_ROW = FEAT // LANES

_mesh = plsc.VectorSubcoreMesh(core_axis_name="c", subcore_axis_name="s")


@functools.partial(
    pl.kernel,
    mesh=_mesh,
    compiler_params=pltpu.CompilerParams(use_tc_tiling_on_sc=False),
    out_type=jax.ShapeDtypeStruct((NUM_WORKERS, LANES), jnp.float32),
    scratch_types=[
        pltpu.VMEM((NCHUNKS, CHUNK), jnp.int32),
        pltpu.VMEM((BPW, FEAT), jnp.float32),
        pltpu.VMEM((BPW, FEAT), jnp.float32),
        pltpu.VMEM((LANES,), jnp.float32),
        pltpu.SemaphoreType.DMA,
    ],
)
def _center_loss_sc(x_hbm, lab_hbm, cen_hbm, out_hbm, idx_v, x_v, c_v,
                    acc_v, sem):
    wid = lax.axis_index("s") * NUM_CORES + lax.axis_index("c")
    base = wid * BPW

    for j in range(NCHUNKS):
        pltpu.sync_copy(lab_hbm.at[pl.ds(base + j * CHUNK, CHUNK)],
                        idx_v.at[j])

    copies = [pltpu.async_copy(x_hbm.at[pl.ds(base, BPW)], x_v, sem)]
    for j in range(NCHUNKS):
        copies.append(
            pltpu.async_copy(cen_hbm.at[idx_v.at[j]],
                             c_v.at[pl.ds(j * CHUNK, CHUNK)], sem))
    for cp in copies:
        cp.wait()

    zero = jnp.zeros((LANES,), jnp.float32)

    def row_body(r, accs):
        new = []
        for v in range(VECS_PER_ROW):
            xv = x_v[r, pl.ds(v * LANES, LANES)]
            cv = c_v[r, pl.ds(v * LANES, LANES)]
            d = xv - cv
            new.append(accs[v] + d * d)
        return tuple(new)

    accs = lax.fori_loop(0, BPW, row_body, (zero,) * VECS_PER_ROW)
    acc_v[...] = (accs[0] + accs[1]) + (accs[2] + accs[3])
    pltpu.sync_copy(acc_v, out_hbm.at[wid])


def kernel(x, labels, centers):
    partials = _center_loss_sc(x, labels.astype(jnp.int32), centers)
    return jnp.sum(partials) / x.shape[0]
